# baseline (speedup 1.0000x reference)
import functools

import jax
import jax.numpy as jnp
from jax import lax
from jax.experimental import pallas as pl
from jax.experimental.pallas import tpu as pltpu

_TEMPERATURE = 0.07
_EPS = 1e-8
_BM = 512
_LANES = 128
_SCALE = 1.4426950408889634 / _TEMPERATURE


def _normalize_kernel(x_ref, out_ref, outs_ref):
    x = x_ref[...]
    ssq = jnp.sum(x * x, axis=1, keepdims=True)
    e = x * lax.rsqrt(ssq)
    out_ref[...] = e.astype(jnp.bfloat16)
    outs_ref[...] = (e * jnp.float32(_SCALE)).astype(jnp.bfloat16)


def _loss_kernel(en_hbm, ens_ref, labr_ref, labc_ref, loss_ref, cnt_ref,
                 en_ref, sem, *, nj, bm):
    i = pl.program_id(0)

    @pl.when(i == 0)
    def _load():
        cp = pltpu.make_async_copy(en_hbm, en_ref, sem)
        cp.start()
        cp.wait()
        loss_ref[...] = jnp.zeros((1, 1), jnp.float32)
        cnt_ref[...] = jnp.zeros((1, 1), jnp.float32)

    lr = labr_ref[...]
    num_acc = jnp.zeros((bm, _LANES), jnp.float32)
    den_acc = jnp.zeros((bm, _LANES), jnp.float32)
    erow = ens_ref[...]
    for t in range(nj):
        j = i if t == 0 else lax.rem(i + t, nj)
        eblk = en_ref[j]
        sim = lax.dot_general(
            erow, eblk, (((1,), (1,)), ((), ())),
            preferred_element_type=jnp.float32)
        lc = labc_ref[j]
        for c in range(bm // _LANES):
            sl = slice(c * _LANES, (c + 1) * _LANES)
            sim_c = sim[:, sl]
            ex_c = jnp.exp2(sim_c)
            pos_c = jnp.where(sim_c > 0, ex_c, 0.0)
            num_c = jnp.where(lr == lc[:, sl], pos_c, 0.0)
            if t == 0:
                rows = lax.broadcasted_iota(jnp.int32, (bm, _LANES), 0)
                cols = lax.broadcasted_iota(jnp.int32, (bm, _LANES), 1) + c * _LANES
                ndiag = rows != cols
                num_c = jnp.where(ndiag, num_c, 0.0)
                den_acc = den_acc + jnp.where(ndiag, ex_c, 0.0)
            else:
                den_acc = den_acc + ex_c
            num_acc = num_acc + num_c
    num_row = jnp.sum(num_acc, axis=1, keepdims=True)
    den_row = jnp.sum(den_acc, axis=1, keepdims=True)
    rvalid = (num_row > 0.0) & (den_row > 0.0)
    num_s = jnp.where(rvalid, num_row, 1.0)
    den_s = jnp.where(rvalid, den_row, 1.0)
    li = -jnp.log(num_s / (den_s + _EPS))
    li = jnp.where(rvalid, li, 0.0)
    loss_ref[...] += jnp.sum(li, axis=0, keepdims=True)
    cnt_ref[...] += jnp.sum(rvalid.astype(jnp.float32), axis=0, keepdims=True)


def kernel(embeddings, labels):
    n, d = embeddings.shape
    bm = _BM if n % _BM == 0 else n
    nj = n // bm

    en, ens = pl.pallas_call(
        _normalize_kernel,
        grid=(nj,),
        in_specs=[pl.BlockSpec((bm, d), lambda i: (i, 0))],
        out_specs=[
            pl.BlockSpec((bm, d), lambda i: (i, 0)),
            pl.BlockSpec((bm, d), lambda i: (i, 0)),
        ],
        out_shape=[
            jax.ShapeDtypeStruct((n, d), jnp.bfloat16),
            jax.ShapeDtypeStruct((n, d), jnp.bfloat16),
        ],
    )(embeddings.astype(jnp.float32))

    labf = labels.astype(jnp.float32)
    labr = jnp.broadcast_to(labf[:, None], (n, _LANES))
    labc = labf.reshape(nj, 1, bm)

    loss_sum, cnt = pl.pallas_call(
        functools.partial(_loss_kernel, nj=nj, bm=bm),
        grid=(nj,),
        in_specs=[
            pl.BlockSpec(memory_space=pl.ANY),
            pl.BlockSpec((bm, d), lambda i: (i, 0)),
            pl.BlockSpec((bm, _LANES), lambda i: (i, 0)),
            pl.BlockSpec((nj, 1, bm), lambda i: (0, 0, 0)),
        ],
        out_specs=[
            pl.BlockSpec((1, 1), lambda i: (0, 0)),
            pl.BlockSpec((1, 1), lambda i: (0, 0)),
        ],
        out_shape=[
            jax.ShapeDtypeStruct((1, 1), jnp.float32),
            jax.ShapeDtypeStruct((1, 1), jnp.float32),
        ],
        scratch_shapes=[
            pltpu.VMEM((nj, bm, d), jnp.bfloat16),
            pltpu.SemaphoreType.DMA,
        ],
        compiler_params=pltpu.CompilerParams(
            dimension_semantics=("arbitrary",),
            vmem_limit_bytes=100 * 1024 * 1024,
        ),
    )(en.reshape(nj, bm, d), ens, labr, labc)

    total = loss_sum[0, 0]
    cnt = cnt[0, 0]
    mean = total / jnp.maximum(cnt, 1.0)
    return jnp.abs(jnp.where(cnt > 0.0, mean, 0.0))


# speedup vs baseline: 1.0091x; 1.0091x over previous
import functools

import jax
import jax.numpy as jnp
from jax import lax
from jax.experimental import pallas as pl
from jax.experimental.pallas import tpu as pltpu

_TEMPERATURE = 0.07
_EPS = 1e-8
_BM = 512
_LANES = 128
_SCALE = 1.4426950408889634 / _TEMPERATURE


def _normalize_kernel(x_ref, outt_ref, outs_ref):
    x = x_ref[...]
    ssq = jnp.sum(x * x, axis=1, keepdims=True)
    e = x * lax.rsqrt(ssq)
    outt_ref[...] = e.T.astype(jnp.bfloat16)[None]
    outs_ref[...] = (e * jnp.float32(_SCALE)).astype(jnp.bfloat16)


def _loss_kernel(en_hbm, ens_ref, labr_ref, labc_ref, loss_ref, cnt_ref,
                 en_ref, sem, *, nj, bm):
    i = pl.program_id(0)

    @pl.when(i == 0)
    def _load():
        cp = pltpu.make_async_copy(en_hbm, en_ref, sem)
        cp.start()
        cp.wait()
        loss_ref[...] = jnp.zeros((1, 1), jnp.float32)
        cnt_ref[...] = jnp.zeros((1, 1), jnp.float32)

    lr = labr_ref[...]
    num_acc = jnp.zeros((bm, _LANES), jnp.float32)
    den_acc = jnp.zeros((bm, _LANES), jnp.float32)
    erow = ens_ref[...]
    for t in range(nj):
        j = i if t == 0 else lax.rem(i + t, nj)
        eblk = en_ref[j]
        sim = lax.dot_general(
            erow, eblk, (((1,), (0,)), ((), ())),
            preferred_element_type=jnp.float32)
        lc = labc_ref[j]
        for c in range(bm // _LANES):
            sl = slice(c * _LANES, (c + 1) * _LANES)
            sim_c = sim[:, sl]
            ex_c = jnp.exp2(sim_c)
            pos_c = jnp.where(sim_c > 0, ex_c, 0.0)
            num_c = jnp.where(lr == lc[:, sl], pos_c, 0.0)
            if t == 0:
                rows = lax.broadcasted_iota(jnp.int32, (bm, _LANES), 0)
                cols = lax.broadcasted_iota(jnp.int32, (bm, _LANES), 1) + c * _LANES
                ndiag = rows != cols
                num_c = jnp.where(ndiag, num_c, 0.0)
                den_acc = den_acc + jnp.where(ndiag, ex_c, 0.0)
            else:
                den_acc = den_acc + ex_c
            num_acc = num_acc + num_c
    num_row = jnp.sum(num_acc, axis=1, keepdims=True)
    den_row = jnp.sum(den_acc, axis=1, keepdims=True)
    rvalid = (num_row > 0.0) & (den_row > 0.0)
    num_s = jnp.where(rvalid, num_row, 1.0)
    den_s = jnp.where(rvalid, den_row, 1.0)
    li = -jnp.log(num_s / (den_s + _EPS))
    li = jnp.where(rvalid, li, 0.0)
    loss_ref[...] += jnp.sum(li, axis=0, keepdims=True)
    cnt_ref[...] += jnp.sum(rvalid.astype(jnp.float32), axis=0, keepdims=True)


def kernel(embeddings, labels):
    n, d = embeddings.shape
    bm = _BM if n % _BM == 0 else n
    nj = n // bm

    ent, ens = pl.pallas_call(
        _normalize_kernel,
        grid=(nj,),
        in_specs=[pl.BlockSpec((bm, d), lambda i: (i, 0))],
        out_specs=[
            pl.BlockSpec((1, d, bm), lambda i: (i, 0, 0)),
            pl.BlockSpec((bm, d), lambda i: (i, 0)),
        ],
        out_shape=[
            jax.ShapeDtypeStruct((nj, d, bm), jnp.bfloat16),
            jax.ShapeDtypeStruct((n, d), jnp.bfloat16),
        ],
    )(embeddings.astype(jnp.float32))

    labf = labels.astype(jnp.float32)
    labr = jnp.broadcast_to(labf[:, None], (n, _LANES))
    labc = labf.reshape(nj, 1, bm)

    loss_sum, cnt = pl.pallas_call(
        functools.partial(_loss_kernel, nj=nj, bm=bm),
        grid=(nj,),
        in_specs=[
            pl.BlockSpec(memory_space=pl.ANY),
            pl.BlockSpec((bm, d), lambda i: (i, 0)),
            pl.BlockSpec((bm, _LANES), lambda i: (i, 0)),
            pl.BlockSpec((nj, 1, bm), lambda i: (0, 0, 0)),
        ],
        out_specs=[
            pl.BlockSpec((1, 1), lambda i: (0, 0)),
            pl.BlockSpec((1, 1), lambda i: (0, 0)),
        ],
        out_shape=[
            jax.ShapeDtypeStruct((1, 1), jnp.float32),
            jax.ShapeDtypeStruct((1, 1), jnp.float32),
        ],
        scratch_shapes=[
            pltpu.VMEM((nj, d, bm), jnp.bfloat16),
            pltpu.SemaphoreType.DMA,
        ],
        compiler_params=pltpu.CompilerParams(
            dimension_semantics=("arbitrary",),
            vmem_limit_bytes=100 * 1024 * 1024,
        ),
    )(ent, ens, labr, labc)

    total = loss_sum[0, 0]
    cnt = cnt[0, 0]
    mean = total / jnp.maximum(cnt, 1.0)
    return jnp.abs(jnp.where(cnt > 0.0, mean, 0.0))
